# baseline (speedup 1.0000x reference)
import functools

import jax
import jax.numpy as jnp
from jax import lax
from jax.experimental import pallas as pl
from jax.experimental.pallas import tpu as pltpu
from jax.experimental.pallas import tpu_sc as plsc

_PARENT = (1, 1, 20, 2, 20, 4, 5, 6, 20, 8, 9, 10, 0, 12, 13, 14, 0, 16,
           17, 18, 1, 7, 7, 11, 11)

_J, _T = 25, 300
_N = 3072
_NW = 32
_PPW = _N // _NW

_CHUNK_OFFS = tuple(range(0, _T - 16, 16)) + (_T - 16,)


def _plane_compute(inb, outb):
    for j in range(_J):
        p = _PARENT[j]
        for off in _CHUNK_OFFS:
            outb[j, pl.ds(off, 16)] = (
                inb[j, pl.ds(off, 16)] - inb[p, pl.ds(off, 16)])


@functools.partial(
    pl.kernel,
    mesh=plsc.VectorSubcoreMesh(core_axis_name="c", subcore_axis_name="s"),
    out_type=jax.ShapeDtypeStruct((_N, _J, _T), jnp.float32),
    scratch_types=[
        pltpu.VMEM((_J, _T), jnp.float32),
        pltpu.VMEM((_J, _T), jnp.float32),
    ],
)
def _sc_joint2bone(x_hbm, out_hbm, inb, outb):
    wid = lax.axis_index("s") * 2 + lax.axis_index("c")
    base = wid * _PPW

    def body(i, _):
        plane = base + i
        pltpu.sync_copy(x_hbm.at[plane], inb)
        _plane_compute(inb, outb)
        pltpu.sync_copy(outb, out_hbm.at[plane])
        return _

    lax.fori_loop(0, _PPW, body, None)


def kernel(joint):
    B, C, J, T = joint.shape
    assert (B * C, J, T) == (_N, _J, _T)
    x = joint.reshape(_N, J, T)
    out = _sc_joint2bone(x)
    return out.reshape(B, C, J, T)


# speedup vs baseline: 1.4369x; 1.4369x over previous
import functools

import jax
import jax.numpy as jnp
from jax import lax
from jax.experimental import pallas as pl
from jax.experimental.pallas import tpu as pltpu
from jax.experimental.pallas import tpu_sc as plsc

_PARENT = (1, 1, 20, 2, 20, 4, 5, 6, 20, 8, 9, 10, 0, 12, 13, 14, 0, 16,
           17, 18, 1, 7, 7, 11, 11)

_J, _T = 25, 300
_N = 3072
_NW = 32
_PPW = _N // _NW
_HALF = _PPW // 2

_CHUNK_OFFS = tuple(range(0, _T - 16, 16)) + (_T - 16,)


def _plane_compute(inb, outb):
    for off in _CHUNK_OFFS:
        rows = [inb[j, pl.ds(off, 16)] for j in range(_J)]
        for j in range(_J):
            outb[j, pl.ds(off, 16)] = rows[j] - rows[_PARENT[j]]


@functools.partial(
    pl.kernel,
    mesh=plsc.VectorSubcoreMesh(core_axis_name="c", subcore_axis_name="s"),
    out_type=jax.ShapeDtypeStruct((_N, _J, _T), jnp.float32),
    scratch_types=[
        pltpu.VMEM((_J, _T), jnp.float32),
        pltpu.VMEM((_J, _T), jnp.float32),
        pltpu.VMEM((_J, _T), jnp.float32),
        pltpu.VMEM((_J, _T), jnp.float32),
        pltpu.SemaphoreType.DMA,
        pltpu.SemaphoreType.DMA,
        pltpu.SemaphoreType.DMA,
        pltpu.SemaphoreType.DMA,
    ],
)
def _sc_joint2bone(x_hbm, out_hbm, in0, in1, ot0, ot1,
                   isem0, isem1, osem0, osem1):
    wid = lax.axis_index("s") * 2 + lax.axis_index("c")
    base = wid * _PPW

    pltpu.async_copy(x_hbm.at[base], in0, isem0)

    def _wait_in(buf, sem):
        pltpu.make_async_copy(x_hbm.at[base], buf, sem).wait()

    def _wait_out(buf, sem):
        pltpu.make_async_copy(buf, out_hbm.at[base], sem).wait()

    def body(t, _):
        cur0 = base + 2 * t

        pltpu.async_copy(x_hbm.at[cur0 + 1], in1, isem1)
        _wait_in(in0, isem0)

        @pl.when(t > 0)
        def _():
            _wait_out(ot0, osem0)

        _plane_compute(in0, ot0)
        pltpu.async_copy(ot0, out_hbm.at[cur0], osem0)

        @pl.when(t < _HALF - 1)
        def _():
            pltpu.async_copy(x_hbm.at[cur0 + 2], in0, isem0)

        _wait_in(in1, isem1)

        @pl.when(t > 0)
        def _():
            _wait_out(ot1, osem1)

        _plane_compute(in1, ot1)
        pltpu.async_copy(ot1, out_hbm.at[cur0 + 1], osem1)
        return _

    lax.fori_loop(0, _HALF, body, None)
    _wait_out(ot0, osem0)
    _wait_out(ot1, osem1)


def kernel(joint):
    B, C, J, T = joint.shape
    assert (B * C, J, T) == (_N, _J, _T)
    x = joint.reshape(_N, J, T)
    out = _sc_joint2bone(x)
    return out.reshape(B, C, J, T)
